# Initial kernel scaffold; baseline (speedup 1.0000x reference)
import jax
import jax.numpy as jnp
from jax.experimental import pallas as pl


def kernel(qkv, position_ids, cos, sin):
    raise NotImplementedError("write your pallas kernel here")


# speedup vs baseline: 1.0096x; 1.0096x over previous
import jax
import jax.numpy as jnp
from jax import lax
from jax.experimental import pallas as pl
from jax.experimental.pallas import tpu as pltpu
from jax.experimental.pallas import tpu_sc as plsc

_INFO = plsc.get_sparse_core_info()
_NC = _INFO.num_cores
_NS = _INFO.num_subcores
_NW = _NC * _NS


def _make_gather(n_idx, dim):
    assert n_idx % (8 * _NW) == 0
    per_w = n_idx // _NW
    mesh = plsc.VectorSubcoreMesh(core_axis_name="c", subcore_axis_name="s")

    @pl.kernel(
        mesh=mesh,
        compiler_params=pltpu.CompilerParams(use_tc_tiling_on_sc=False),
        out_type=(
            jax.ShapeDtypeStruct((n_idx, dim), jnp.float32),
            jax.ShapeDtypeStruct((n_idx, dim), jnp.float32),
        ),
        scratch_types=[
            pltpu.VMEM((per_w,), jnp.int32),
            pltpu.VMEM((per_w, dim), jnp.float32),
            pltpu.VMEM((per_w, dim), jnp.float32),
            pltpu.SemaphoreType.DMA,
            pltpu.SemaphoreType.DMA,
        ],
    )
    def gather_kernel(pos_hbm, cos_hbm, sin_hbm, cos_out, sin_out,
                      idx_v, cos_v, sin_v, sem_c, sem_s):
        wid = lax.axis_index("s") * _NC + lax.axis_index("c")
        base = wid * per_w
        pltpu.sync_copy(pos_hbm.at[pl.ds(base, per_w)], idx_v)
        cpy_c = pltpu.async_copy(cos_hbm.at[idx_v], cos_v, sem_c)
        cpy_s = pltpu.async_copy(sin_hbm.at[idx_v], sin_v, sem_s)
        cpy_c.wait()
        pltpu.sync_copy(cos_v, cos_out.at[pl.ds(base, per_w)])
        cpy_s.wait()
        pltpu.sync_copy(sin_v, sin_out.at[pl.ds(base, per_w)])

    return gather_kernel


def kernel(qkv, position_ids, cos, sin):
    b, s = position_ids.shape
    dim = cos.shape[1]
    flat_ids = position_ids.reshape(b * s).astype(jnp.int32)
    cos_rows, sin_rows = _make_gather(b * s, dim)(flat_ids, cos, sin)
    return cos_rows.reshape(b, s, dim), sin_rows.reshape(b, s, dim)


# speedup vs baseline: 1.0119x; 1.0023x over previous
import jax
import jax.numpy as jnp
from jax import lax
from jax.experimental import pallas as pl
from jax.experimental.pallas import tpu as pltpu
from jax.experimental.pallas import tpu_sc as plsc

_INFO = plsc.get_sparse_core_info()
_NC = _INFO.num_cores
_NS = _INFO.num_subcores
_NW = _NC * _NS


def _make_gather(n_idx, dim):
    assert n_idx % (8 * _NW) == 0
    per_w = n_idx // _NW
    mesh = plsc.VectorSubcoreMesh(core_axis_name="c", subcore_axis_name="s")

    @pl.kernel(
        mesh=mesh,
        compiler_params=pltpu.CompilerParams(use_tc_tiling_on_sc=False),
        out_type=(
            jax.ShapeDtypeStruct((n_idx, dim), jnp.float32),
            jax.ShapeDtypeStruct((n_idx, dim), jnp.float32),
        ),
        scratch_types=[
            pltpu.VMEM((per_w,), jnp.int32),
            pltpu.VMEM((per_w, dim), jnp.float32),
            pltpu.VMEM((per_w, dim), jnp.float32),
            pltpu.SemaphoreType.DMA,
            pltpu.SemaphoreType.DMA,
            pltpu.SemaphoreType.DMA,
            pltpu.SemaphoreType.DMA,
        ],
    )
    def gather_kernel(pos_hbm, cos_hbm, sin_hbm, cos_out, sin_out,
                      idx_v, cos_v, sin_v, sem_c, sem_s, sem_oc, sem_os):
        wid = lax.axis_index("s") * _NC + lax.axis_index("c")
        base = wid * per_w
        pltpu.sync_copy(pos_hbm.at[pl.ds(base, per_w)], idx_v)
        cpy_c = pltpu.async_copy(cos_hbm.at[idx_v], cos_v, sem_c)
        cpy_s = pltpu.async_copy(sin_hbm.at[idx_v], sin_v, sem_s)
        cpy_c.wait()
        out_c = pltpu.async_copy(cos_v, cos_out.at[pl.ds(base, per_w)], sem_oc)
        cpy_s.wait()
        out_s = pltpu.async_copy(sin_v, sin_out.at[pl.ds(base, per_w)], sem_os)
        out_c.wait()
        out_s.wait()

    return gather_kernel


def kernel(qkv, position_ids, cos, sin):
    b, s = position_ids.shape
    dim = cos.shape[1]
    flat_ids = position_ids.reshape(b * s).astype(jnp.int32)
    cos_rows, sin_rows = _make_gather(b * s, dim)(flat_ids, cos, sin)
    return cos_rows.reshape(b, s, dim), sin_rows.reshape(b, s, dim)


# speedup vs baseline: 1.0129x; 1.0009x over previous
import jax
import jax.numpy as jnp
from jax import lax
from jax.experimental import pallas as pl
from jax.experimental.pallas import tpu as pltpu
from jax.experimental.pallas import tpu_sc as plsc

_INFO = plsc.get_sparse_core_info()
_NC = _INFO.num_cores
_NS = _INFO.num_subcores
_NW = _NC * _NS


def _make_gather(b, s, dim):
    n_idx = b * s
    assert n_idx % (8 * _NW) == 0
    per_w = n_idx // _NW
    assert s % per_w == 0
    mesh = plsc.VectorSubcoreMesh(core_axis_name="c", subcore_axis_name="s")

    @pl.kernel(
        mesh=mesh,
        compiler_params=pltpu.CompilerParams(use_tc_tiling_on_sc=False),
        out_type=(
            jax.ShapeDtypeStruct((b, s, dim), jnp.float32),
            jax.ShapeDtypeStruct((b, s, dim), jnp.float32),
        ),
        scratch_types=[
            pltpu.VMEM((per_w,), jnp.int32),
            pltpu.VMEM((per_w, dim), jnp.float32),
            pltpu.VMEM((per_w, dim), jnp.float32),
            pltpu.SemaphoreType.DMA,
            pltpu.SemaphoreType.DMA,
            pltpu.SemaphoreType.DMA,
            pltpu.SemaphoreType.DMA,
        ],
    )
    def gather_kernel(pos_hbm, cos_hbm, sin_hbm, cos_out, sin_out,
                      idx_v, cos_v, sin_v, sem_c, sem_s, sem_oc, sem_os):
        wid = lax.axis_index("s") * _NC + lax.axis_index("c")
        base = wid * per_w
        bi = base // s
        ri = base % s
        pltpu.sync_copy(pos_hbm.at[bi, pl.ds(ri, per_w)], idx_v)
        cpy_c = pltpu.async_copy(cos_hbm.at[idx_v], cos_v, sem_c)
        cpy_s = pltpu.async_copy(sin_hbm.at[idx_v], sin_v, sem_s)
        cpy_c.wait()
        out_c = pltpu.async_copy(cos_v, cos_out.at[bi, pl.ds(ri, per_w)],
                                 sem_oc)
        cpy_s.wait()
        out_s = pltpu.async_copy(sin_v, sin_out.at[bi, pl.ds(ri, per_w)],
                                 sem_os)
        out_c.wait()
        out_s.wait()

    return gather_kernel


def kernel(qkv, position_ids, cos, sin):
    b, s = position_ids.shape
    dim = cos.shape[1]
    return _make_gather(b, s, dim)(position_ids.astype(jnp.int32), cos, sin)
